# Initial kernel scaffold; baseline (speedup 1.0000x reference)
import jax
import jax.numpy as jnp
from jax.experimental import pallas as pl


def kernel(x_visit, x_drug, Wl, Wr, bconv, gamma, beta, lin1_W, lin1_b, lin2_W, lin2_b, edge_label, edge_index_vd, edge_index_dv, edge_label_index, mask):
    raise NotImplementedError("write your pallas kernel here")


# speedup vs baseline: 1.1433x; 1.1433x over previous
import functools

import jax
import jax.numpy as jnp
from jax.experimental import pallas as pl
from jax.experimental.pallas import tpu as pltpu

_NV, _ND, _E, _L, _H = 50000, 5000, 800000, 100000, 64


def _sage_body(agg_ref, cnt_ref, x_ref, wl_ref, wr_ref, b_ref,
               y_ref, s1_ref, s2_ref, acc1, acc2):
    i = pl.program_id(0)
    cnt = jnp.maximum(cnt_ref[...], 1.0)
    t = agg_ref[...] / cnt
    y = (jnp.dot(t, wl_ref[...], preferred_element_type=jnp.float32)
         + b_ref[...]
         + jnp.dot(x_ref[...], wr_ref[...], preferred_element_type=jnp.float32))
    y = jnp.where(y >= 0.0, y, 0.01 * y)
    y_ref[...] = y

    @pl.when(i == 0)
    def _init():
        acc1[...] = jnp.zeros_like(acc1)
        acc2[...] = jnp.zeros_like(acc2)

    acc1[...] += jnp.sum(y, axis=0, keepdims=True)
    acc2[...] += jnp.sum(y * y, axis=0, keepdims=True)

    @pl.when(i == pl.num_programs(0) - 1)
    def _flush():
        s1_ref[...] = acc1[...]
        s2_ref[...] = acc2[...]


def _sage_stage(agg, cnt, x, wl, wr, b, blk):
    n = x.shape[0]
    grid = n // blk
    y, s1, s2 = pl.pallas_call(
        _sage_body,
        grid=(grid,),
        in_specs=[
            pl.BlockSpec((blk, _H), lambda i: (i, 0)),
            pl.BlockSpec((blk, 1), lambda i: (i, 0)),
            pl.BlockSpec((blk, _H), lambda i: (i, 0)),
            pl.BlockSpec((_H, _H), lambda i: (0, 0)),
            pl.BlockSpec((_H, _H), lambda i: (0, 0)),
            pl.BlockSpec((1, _H), lambda i: (0, 0)),
        ],
        out_specs=[
            pl.BlockSpec((blk, _H), lambda i: (i, 0)),
            pl.BlockSpec((1, _H), lambda i: (0, 0)),
            pl.BlockSpec((1, _H), lambda i: (0, 0)),
        ],
        out_shape=[
            jax.ShapeDtypeStruct((n, _H), jnp.float32),
            jax.ShapeDtypeStruct((1, _H), jnp.float32),
            jax.ShapeDtypeStruct((1, _H), jnp.float32),
        ],
        scratch_shapes=[
            pltpu.VMEM((1, _H), jnp.float32),
            pltpu.VMEM((1, _H), jnp.float32),
        ],
    )(agg, cnt, x, wl, wr, b)
    return y, s1, s2


def _bn_body(y_ref, s1_ref, s2_ref, g_ref, bt_ref, o_ref, *, n):
    m = s1_ref[...] / n
    v = s2_ref[...] / n - m * m
    inv = jax.lax.rsqrt(v + 1e-5)
    o_ref[...] = (y_ref[...] - m) * inv * g_ref[...] + bt_ref[...]


def _bn_proj_body(y_ref, s1_ref, s2_ref, g_ref, bt_ref, pw_ref, pb_ref,
                  o_ref, *, n):
    m = s1_ref[...] / n
    v = s2_ref[...] / n - m * m
    inv = jax.lax.rsqrt(v + 1e-5)
    h = (y_ref[...] - m) * inv * g_ref[...] + bt_ref[...]
    o_ref[...] = (jnp.dot(h, pw_ref[...], preferred_element_type=jnp.float32)
                  + pb_ref[...])


def _bn_stage(y, s1, s2, g, bt, blk, proj=None):
    n = y.shape[0]
    grid = n // blk
    row = pl.BlockSpec((blk, _H), lambda i: (i, 0))
    stat = pl.BlockSpec((1, _H), lambda i: (0, 0))
    if proj is None:
        body = functools.partial(_bn_body, n=float(n))
        args = (y, s1, s2, g, bt)
        specs = [row, stat, stat, stat, stat]
    else:
        pw, pb = proj
        body = functools.partial(_bn_proj_body, n=float(n))
        args = (y, s1, s2, g, bt, pw, pb)
        specs = [row, stat, stat, stat, stat,
                 pl.BlockSpec((_H, _H), lambda i: (0, 0)), stat]
    return pl.pallas_call(
        body,
        grid=(grid,),
        in_specs=specs,
        out_specs=row,
        out_shape=jax.ShapeDtypeStruct((n, _H), jnp.float32),
    )(*args)


def _mlp_body(s_ref, w2_ref, b2_ref, o_ref):
    z = jnp.maximum(s_ref[...], 0.0)
    acc = jnp.sum(z * w2_ref[...], axis=1, keepdims=True) + b2_ref[...]
    o_ref[...] = jax.nn.sigmoid(acc)


def _mlp_stage(s, w2, b2, blk):
    n = s.shape[0]
    return pl.pallas_call(
        _mlp_body,
        grid=(n // blk,),
        in_specs=[
            pl.BlockSpec((blk, _H), lambda i: (i, 0)),
            pl.BlockSpec((1, _H), lambda i: (0, 0)),
            pl.BlockSpec((1, 1), lambda i: (0, 0)),
        ],
        out_specs=pl.BlockSpec((blk, 1), lambda i: (i, 0)),
        out_shape=jax.ShapeDtypeStruct((n, 1), jnp.float32),
    )(s, w2, b2)


def _seg_sum(x_src, ei, n_dst):
    msg = jnp.take(x_src, ei[0], axis=0)
    return jax.ops.segment_sum(msg, ei[1], num_segments=n_dst)


def kernel(x_visit, x_drug, Wl, Wr, bconv, gamma, beta, lin1_W, lin1_b,
           lin2_W, lin2_b, edge_label, edge_index_vd, edge_index_dv,
           edge_label_index, mask):
    ones = jnp.ones((_E,), jnp.float32)
    cnt_d = jax.ops.segment_sum(ones, edge_index_vd[1], num_segments=_ND)
    cnt_v = jax.ops.segment_sum(ones, edge_index_dv[1], num_segments=_NV)
    cnt_d = cnt_d.reshape(_ND, 1)
    cnt_v = cnt_v.reshape(_NV, 1)

    hv, hd = x_visit, x_drug
    w1a = lin1_W[:_H]
    w1b = lin1_W[_H:]
    for l in range(4):
        agg_d = _seg_sum(hv, edge_index_vd, _ND)
        agg_v = _seg_sum(hd, edge_index_dv, _NV)
        yd, sd1, sd2 = _sage_stage(agg_d, cnt_d, hd, Wl[l, 0], Wr[l, 0],
                                   bconv[l, 0].reshape(1, _H), 5000)
        yv, sv1, sv2 = _sage_stage(agg_v, cnt_v, hv, Wl[l, 1], Wr[l, 1],
                                   bconv[l, 1].reshape(1, _H), 5000)
        gv = gamma[l, 0].reshape(1, _H)
        bv = beta[l, 0].reshape(1, _H)
        gd = gamma[l, 1].reshape(1, _H)
        bd = beta[l, 1].reshape(1, _H)
        if l < 3:
            hv = _bn_stage(yv, sv1, sv2, gv, bv, 5000)
            hd = _bn_stage(yd, sd1, sd2, gd, bd, 5000)
        else:
            zero = jnp.zeros((1, _H), jnp.float32)
            pv = _bn_stage(yv, sv1, sv2, gv, bv, 5000, proj=(w1a, zero))
            pd = _bn_stage(yd, sd1, sd2, gd, bd, 5000,
                           proj=(w1b, lin1_b.reshape(1, _H)))

    s = (jnp.take(pv, edge_label_index[0], axis=0)
         + jnp.take(pd, edge_label_index[1], axis=0))
    out = _mlp_stage(s, lin2_W.reshape(1, _H), lin2_b.reshape(1, 1), 5000)
    return out.reshape(-1)
